# baseline (speedup 1.0000x reference)
import numpy as np
import jax
import jax.numpy as jnp
from jax.experimental import pallas as pl
from jax.experimental.pallas import tpu as pltpu

_F32 = jnp.float32
_BF16 = jnp.bfloat16
_HIGHEST = jax.lax.Precision.HIGHEST


def _weight_mat(in_size: int, out_size: int) -> np.ndarray:
    scale = np.float32(out_size) / np.float32(in_size)
    inv = np.float32(1.0) / scale
    ks = np.maximum(inv, np.float32(1.0))
    sample_f = (np.arange(out_size, dtype=np.float32) + np.float32(0.5)) * inv - np.float32(0.5)
    x = np.abs(sample_f[None, :] - np.arange(in_size, dtype=np.float32)[:, None]) / ks
    w = np.maximum(np.float32(0), np.float32(1) - x).astype(np.float32)
    tot = w.sum(axis=0, keepdims=True, dtype=np.float32)
    w = np.where(np.abs(tot) > 1000 * np.finfo(np.float32).eps,
                 w / np.where(tot != 0, tot, 1), 0).astype(np.float32)
    valid = (sample_f >= -0.5) & (sample_f <= np.float32(in_size) - 0.5)
    return np.where(valid[None, :], w, 0).astype(np.float32)


_M16 = _weight_mat(32, 16)
_M8 = _weight_mat(32, 8)
_K16 = np.kron(_M16, _M16)
_K8 = np.kron(_M8, _M8)
_BD16 = np.kron(np.eye(8, dtype=np.float32), _M16)
_BD8 = np.kron(np.eye(8, dtype=np.float32), _M8)


def _bf16_split(w: np.ndarray):
    import ml_dtypes
    hi = w.astype(ml_dtypes.bfloat16)
    lo = (w - hi.astype(np.float32)).astype(ml_dtypes.bfloat16)
    return hi, lo


_K16HI, _K16LO = _bf16_split(_K16)
_K8HI, _K8LO = _bf16_split(_K8)
_BD16HI, _BD16LO = _bf16_split(_BD16)
_BD8HI, _BD8LO = _bf16_split(_BD8)


def _mm(a, b, precision):
    return jax.lax.dot_general(a, b, (((1,), (0,)), ((), ())),
                               precision=precision, preferred_element_type=_F32)


def _mm3(x, whi, wlo):
    xhi = x.astype(_BF16)
    xlo = (x - xhi.astype(_F32)).astype(_BF16)
    return (_mm(xhi, whi, None) + _mm(xhi, wlo, None)) + _mm(xlo, whi, None)


def _resize_one(x_ref, k16hi, k16lo, k8hi, k8lo, bd16hi, bd16lo,
                bd8hi, bd8lo, o16_ref, o8_ref):
    x2 = x_ref[...].reshape(256, 1024)
    z16 = _mm3(x2, k16hi, k16lo)
    z8 = _mm3(x2, k8hi, k8lo)
    y16 = _mm3(z16.T, bd16hi, bd16lo)
    y8 = _mm3(z8.T, bd8hi, bd8lo)
    o16_ref[...] = y16.T.reshape(8, 16, 256)
    o8_ref[...] = y8.T.reshape(8, 8, 64)


def _resize_kernel(xs_ref, xt_ref, k16hi_ref, k16lo_ref, k8hi_ref, k8lo_ref,
                   bd16hi_ref, bd16lo_ref, bd8hi_ref, bd8lo_ref,
                   ts0_ref, tt0_ref, ts1_ref, tt1_ref):
    ws = (k16hi_ref[:], k16lo_ref[:], k8hi_ref[:], k8lo_ref[:],
          bd16hi_ref[:], bd16lo_ref[:], bd8hi_ref[:], bd8lo_ref[:])
    _resize_one(xs_ref, *ws, ts0_ref, ts1_ref)
    _resize_one(xt_ref, *ws, tt0_ref, tt1_ref)


def _min_idx_from_g(ttb, g, blk_rows):
    b2b = jnp.sum(ttb * ttb, axis=1, keepdims=True)
    u = b2b - 2.0 * g
    bmin = jnp.min(u, axis=0, keepdims=True)
    miota = jax.lax.broadcasted_iota(jnp.int32, (blk_rows, 1), 0)
    bidx = jnp.min(jnp.where(u == bmin, miota, 2**30), axis=0, keepdims=True)
    return bmin, bidx


def _block_min_idx(tscm, ttb, blk_rows, n_tok):
    return _min_idx_from_g(ttb, _mm(ttb, tscm, None), blk_rows)


def _decode_loss(idx, et, shift, mask):
    gd = (idx >> (2 * shift)).astype(_F32)
    gh = ((idx >> shift) & mask).astype(_F32)
    gw = (idx & mask).astype(_F32)
    return (jnp.sum(jnp.abs(et[0:1, :] - gd)) +
            jnp.sum(jnp.abs(et[1:2, :] - gh)) +
            jnp.sum(jnp.abs(et[2:3, :] - gw)))


def _loss_kernel(ts0_ref, tt0_ref, e0_ref, ts1_ref, tt1_ref, e1_ref,
                 out_ref, runmin_ref, runidx_ref, tt0tm_ref, tt1tm_ref,
                 e0t_ref, e1t_ref, g2_ref):
    i = pl.program_id(0)
    nprog = pl.num_programs(0)

    @pl.when(i == 0)
    def _prep():
        tt0tm_ref[:] = tt0_ref[:].T
        tt1tm_ref[:] = tt1_ref[:].T
        e0t_ref[:] = e0_ref[:].T
        e1t_ref[:] = e1_ref[:].T
        g2_ref[0] = _mm(tt0tm_ref[0:512, :], ts0_ref[:], None)

    parity = jax.lax.rem(i, 2)
    ttb = tt0tm_ref[pl.ds(i * 512, 512), :]
    g = g2_ref[pl.ds(parity, 1), :, :].reshape(512, 4096)
    nxt = jnp.minimum(i + 1, nprog - 1)
    gnxt = _mm(tt0tm_ref[pl.ds(nxt * 512, 512), :], ts0_ref[:], None)
    g2_ref[pl.ds(1 - parity, 1), :, :] = gnxt.reshape(1, 512, 4096)
    bmin, bidx = _min_idx_from_g(ttb, g, 512)
    bidx = bidx + i * 512

    @pl.when(i == 0)
    def _init():
        runmin_ref[:] = bmin
        runidx_ref[:] = bidx

    @pl.when(i > 0)
    def _merge():
        upd = bmin < runmin_ref[:]
        runidx_ref[:] = jnp.where(upd, bidx, runidx_ref[:])
        runmin_ref[:] = jnp.minimum(bmin, runmin_ref[:])

    @pl.when(i == pl.num_programs(0) - 1)
    def _finalize():
        s0 = _decode_loss(runidx_ref[:], e0t_ref[:], 4, 15)
        _, idx1 = _block_min_idx(ts1_ref[:], tt1tm_ref[:], 512, 512)
        s1 = _decode_loss(idx1, e1t_ref[:], 3, 7)
        out_ref[:, :] = (0.5 * (s0 / (4096.0 * 3.0)
                                + s1 / (512.0 * 3.0))).reshape(1, 1)


def kernel(canonical_source, canonical_target, raw_displacement_0,
           expected_target_positions_0, raw_displacement_1,
           expected_target_positions_1):
    del raw_displacement_0, raw_displacement_1
    xs = canonical_source.reshape(64, 32, 1024)
    xt = canonical_target.reshape(64, 32, 1024)

    full = lambda shape: pl.BlockSpec(shape, lambda i: tuple(0 for _ in shape))
    ts0, tt0, ts1, tt1 = pl.pallas_call(
        _resize_kernel,
        grid=(8,),
        in_specs=[pl.BlockSpec((8, 32, 1024), lambda i: (i, 0, 0)),
                  pl.BlockSpec((8, 32, 1024), lambda i: (i, 0, 0)),
                  full((1024, 256)), full((1024, 256)),
                  full((1024, 64)), full((1024, 64)),
                  full((256, 128)), full((256, 128)),
                  full((256, 64)), full((256, 64))],
        out_specs=(pl.BlockSpec((8, 16, 256), lambda i: (i, 0, 0)),
                   pl.BlockSpec((8, 16, 256), lambda i: (i, 0, 0)),
                   pl.BlockSpec((8, 8, 64), lambda i: (i, 0, 0)),
                   pl.BlockSpec((8, 8, 64), lambda i: (i, 0, 0))),
        out_shape=(jax.ShapeDtypeStruct((64, 16, 256), _F32),
                   jax.ShapeDtypeStruct((64, 16, 256), _F32),
                   jax.ShapeDtypeStruct((64, 8, 64), _F32),
                   jax.ShapeDtypeStruct((64, 8, 64), _F32)),
    )(xs, xt, jnp.asarray(_K16HI), jnp.asarray(_K16LO),
      jnp.asarray(_K8HI), jnp.asarray(_K8LO),
      jnp.asarray(_BD16HI), jnp.asarray(_BD16LO),
      jnp.asarray(_BD8HI), jnp.asarray(_BD8LO))

    out = pl.pallas_call(
        _loss_kernel,
        grid=(8,),
        in_specs=[full((64, 4096)), full((64, 4096)), full((4096, 3)),
                  full((64, 512)), full((64, 512)), full((512, 3))],
        out_specs=pl.BlockSpec((1, 1), lambda i: (0, 0)),
        out_shape=jax.ShapeDtypeStruct((1, 1), _F32),
        scratch_shapes=[pltpu.VMEM((1, 4096), _F32),
                        pltpu.VMEM((1, 4096), jnp.int32),
                        pltpu.VMEM((4096, 64), _F32),
                        pltpu.VMEM((512, 64), _F32),
                        pltpu.VMEM((3, 4096), _F32),
                        pltpu.VMEM((3, 512), _F32),
                        pltpu.VMEM((2, 512, 4096), _F32)],
    )(ts0.reshape(64, 4096), tt0.reshape(64, 4096),
      expected_target_positions_0.reshape(4096, 3),
      ts1.reshape(64, 512), tt1.reshape(64, 512),
      expected_target_positions_1.reshape(512, 3))
    return out[0, 0]


# speedup vs baseline: 1.2246x; 1.2246x over previous
import numpy as np
import jax
import jax.numpy as jnp
from jax.experimental import pallas as pl
from jax.experimental.pallas import tpu as pltpu

_F32 = jnp.float32
_BF16 = jnp.bfloat16
_HIGHEST = jax.lax.Precision.HIGHEST


def _weight_mat(in_size: int, out_size: int) -> np.ndarray:
    scale = np.float32(out_size) / np.float32(in_size)
    inv = np.float32(1.0) / scale
    ks = np.maximum(inv, np.float32(1.0))
    sample_f = (np.arange(out_size, dtype=np.float32) + np.float32(0.5)) * inv - np.float32(0.5)
    x = np.abs(sample_f[None, :] - np.arange(in_size, dtype=np.float32)[:, None]) / ks
    w = np.maximum(np.float32(0), np.float32(1) - x).astype(np.float32)
    tot = w.sum(axis=0, keepdims=True, dtype=np.float32)
    w = np.where(np.abs(tot) > 1000 * np.finfo(np.float32).eps,
                 w / np.where(tot != 0, tot, 1), 0).astype(np.float32)
    valid = (sample_f >= -0.5) & (sample_f <= np.float32(in_size) - 0.5)
    return np.where(valid[None, :], w, 0).astype(np.float32)


_M16 = _weight_mat(32, 16)
_M8 = _weight_mat(32, 8)
_K16 = np.kron(_M16, _M16)
_K8 = np.kron(_M8, _M8)
_BD16 = np.kron(np.eye(16, dtype=np.float32), _M16)
_BD8 = np.kron(np.eye(16, dtype=np.float32), _M8)


def _bf16_split(w: np.ndarray):
    import ml_dtypes
    hi = w.astype(ml_dtypes.bfloat16)
    lo = (w - hi.astype(np.float32)).astype(ml_dtypes.bfloat16)
    return hi, lo


_K16HI, _K16LO = _bf16_split(_K16)
_K8HI, _K8LO = _bf16_split(_K8)
_BD16HI, _BD16LO = _bf16_split(_BD16)
_BD8HI, _BD8LO = _bf16_split(_BD8)


def _mm(a, b, precision):
    return jax.lax.dot_general(a, b, (((1,), (0,)), ((), ())),
                               precision=precision, preferred_element_type=_F32)


def _split(x):
    xhi = x.astype(_BF16)
    xlo = (x - xhi.astype(_F32)).astype(_BF16)
    return xhi, xlo


def _mm3s(xhi, xlo, whi, wlo):
    return (_mm(xhi, whi, None) + _mm(xhi, wlo, None)) + _mm(xlo, whi, None)


def _mm3(x, whi, wlo):
    return _mm3s(*_split(x), whi, wlo)


def _resize_one(x_ref, k16hi, k16lo, k8hi, k8lo, bd16hi, bd16lo,
                bd8hi, bd8lo, o16_ref, o8_ref):
    x2 = x_ref[...].reshape(512, 1024)
    xhi, xlo = _split(x2)
    z16 = _mm3s(xhi, xlo, k16hi, k16lo)
    z8 = _mm3s(xhi, xlo, k8hi, k8lo)
    y16 = _mm3(z16.T, bd16hi, bd16lo)
    y8 = _mm3(z8.T, bd8hi, bd8lo)
    o16_ref[...] = y16.T.reshape(16, 16, 256)
    o8_ref[...] = y8.T.reshape(16, 8, 64)


def _resize_kernel(xs_ref, xt_ref, k16hi_ref, k16lo_ref, k8hi_ref, k8lo_ref,
                   bd16hi_ref, bd16lo_ref, bd8hi_ref, bd8lo_ref,
                   ts0_ref, tt0_ref, ts1_ref, tt1_ref):
    ws = (k16hi_ref[:], k16lo_ref[:], k8hi_ref[:], k8lo_ref[:],
          bd16hi_ref[:], bd16lo_ref[:], bd8hi_ref[:], bd8lo_ref[:])
    _resize_one(xs_ref, *ws, ts0_ref, ts1_ref)
    _resize_one(xt_ref, *ws, tt0_ref, tt1_ref)


def _min_idx_from_g(ttb, g, blk_rows):
    b2b = jnp.sum(ttb * ttb, axis=1, keepdims=True)
    u = b2b - 2.0 * g
    bmin = jnp.min(u, axis=0, keepdims=True)
    miota = jax.lax.broadcasted_iota(jnp.int32, (blk_rows, 1), 0)
    bidx = jnp.min(jnp.where(u == bmin, miota, 2**30), axis=0, keepdims=True)
    return bmin, bidx


def _block_min_idx(tscm, ttb, blk_rows, n_tok):
    return _min_idx_from_g(ttb, _mm(ttb, tscm, None), blk_rows)


def _decode_loss(idx, et, shift, mask):
    gd = (idx >> (2 * shift)).astype(_F32)
    gh = ((idx >> shift) & mask).astype(_F32)
    gw = (idx & mask).astype(_F32)
    return (jnp.sum(jnp.abs(et[0:1, :] - gd)) +
            jnp.sum(jnp.abs(et[1:2, :] - gh)) +
            jnp.sum(jnp.abs(et[2:3, :] - gw)))


def _loss_kernel(ts0_ref, tt0_ref, e0_ref, ts1_ref, tt1_ref, e1_ref,
                 out_ref, runmin_ref, runidx_ref, tt0tm_ref, tt1tm_ref,
                 e0t_ref, e1t_ref):
    i = pl.program_id(0)

    @pl.when(i == 0)
    def _prep():
        tt0tm_ref[:] = tt0_ref[:].T
        tt1tm_ref[:] = tt1_ref[:].T
        e0t_ref[:] = e0_ref[:].T
        e1t_ref[:] = e1_ref[:].T

    ttb = tt0tm_ref[pl.ds(i * 512, 512), :]
    bmin, bidx = _block_min_idx(ts0_ref[:], ttb, 512, 4096)
    bidx = bidx + i * 512

    @pl.when(i == 0)
    def _init():
        runmin_ref[:] = bmin
        runidx_ref[:] = bidx

    @pl.when(i > 0)
    def _merge():
        upd = bmin < runmin_ref[:]
        runidx_ref[:] = jnp.where(upd, bidx, runidx_ref[:])
        runmin_ref[:] = jnp.minimum(bmin, runmin_ref[:])

    @pl.when(i == pl.num_programs(0) - 1)
    def _finalize():
        s0 = _decode_loss(runidx_ref[:], e0t_ref[:], 4, 15)
        _, idx1 = _block_min_idx(ts1_ref[:], tt1tm_ref[:], 512, 512)
        s1 = _decode_loss(idx1, e1t_ref[:], 3, 7)
        out_ref[:, :] = (0.5 * (s0 / (4096.0 * 3.0)
                                + s1 / (512.0 * 3.0))).reshape(1, 1)


def kernel(canonical_source, canonical_target, raw_displacement_0,
           expected_target_positions_0, raw_displacement_1,
           expected_target_positions_1):
    del raw_displacement_0, raw_displacement_1
    xs = canonical_source.reshape(64, 32, 1024)
    xt = canonical_target.reshape(64, 32, 1024)

    full = lambda shape: pl.BlockSpec(shape, lambda i: tuple(0 for _ in shape))
    ts0, tt0, ts1, tt1 = pl.pallas_call(
        _resize_kernel,
        grid=(4,),
        in_specs=[pl.BlockSpec((16, 32, 1024), lambda i: (i, 0, 0)),
                  pl.BlockSpec((16, 32, 1024), lambda i: (i, 0, 0)),
                  full((1024, 256)), full((1024, 256)),
                  full((1024, 64)), full((1024, 64)),
                  full((512, 256)), full((512, 256)),
                  full((512, 128)), full((512, 128))],
        out_specs=(pl.BlockSpec((16, 16, 256), lambda i: (i, 0, 0)),
                   pl.BlockSpec((16, 16, 256), lambda i: (i, 0, 0)),
                   pl.BlockSpec((16, 8, 64), lambda i: (i, 0, 0)),
                   pl.BlockSpec((16, 8, 64), lambda i: (i, 0, 0))),
        out_shape=(jax.ShapeDtypeStruct((64, 16, 256), _F32),
                   jax.ShapeDtypeStruct((64, 16, 256), _F32),
                   jax.ShapeDtypeStruct((64, 8, 64), _F32),
                   jax.ShapeDtypeStruct((64, 8, 64), _F32)),
    )(xs, xt, jnp.asarray(_K16HI), jnp.asarray(_K16LO),
      jnp.asarray(_K8HI), jnp.asarray(_K8LO),
      jnp.asarray(_BD16HI), jnp.asarray(_BD16LO),
      jnp.asarray(_BD8HI), jnp.asarray(_BD8LO))

    out = pl.pallas_call(
        _loss_kernel,
        grid=(8,),
        in_specs=[full((64, 4096)), full((64, 4096)), full((4096, 3)),
                  full((64, 512)), full((64, 512)), full((512, 3))],
        out_specs=pl.BlockSpec((1, 1), lambda i: (0, 0)),
        out_shape=jax.ShapeDtypeStruct((1, 1), _F32),
        scratch_shapes=[pltpu.VMEM((1, 4096), _F32),
                        pltpu.VMEM((1, 4096), jnp.int32),
                        pltpu.VMEM((4096, 64), _F32),
                        pltpu.VMEM((512, 64), _F32),
                        pltpu.VMEM((3, 4096), _F32),
                        pltpu.VMEM((3, 512), _F32)],
    )(ts0.reshape(64, 4096), tt0.reshape(64, 4096),
      expected_target_positions_0.reshape(4096, 3),
      ts1.reshape(64, 512), tt1.reshape(64, 512),
      expected_target_positions_1.reshape(512, 3))
    return out[0, 0]
